# baseline (speedup 1.0000x reference)
import functools

import jax
import jax.numpy as jnp
from jax import lax
from jax.experimental import pallas as pl
from jax.experimental.pallas import tpu as pltpu
from jax.experimental.pallas import tpu_sc as plsc

EMBED = 16
FIELDS_USED = 9
LANE = 128


@functools.cache
def _make_gather(batch: int, vocab: int):
    nc, ns = 2, 16
    d_per_c = EMBED // nc
    units = FIELDS_USED * d_per_c * 2
    u_per_w = units // ns
    half = batch // 2

    chunk_i = 64 * LANE
    n_full = vocab // chunk_i
    extra_i = n_full * chunk_i
    extra_sz = (vocab - extra_i) // LANE * LANE
    tail_i = extra_i + extra_sz
    tail_sz = vocab - tail_i
    mesh = plsc.VectorSubcoreMesh(core_axis_name="c", subcore_axis_name="s")

    @functools.partial(
        pl.kernel,
        mesh=mesh,
        out_type=jax.ShapeDtypeStruct((FIELDS_USED, EMBED, batch), jnp.float32),
        scratch_types=[
            pltpu.HBM((EMBED, vocab), jnp.float32),
            pltpu.HBM((EMBED * vocab,), jnp.float32),
            pltpu.VMEM((chunk_i,), jnp.float32),
            pltpu.VMEM((tail_sz,), jnp.float32),
            pltpu.VMEM((half,), jnp.int32),
            pltpu.VMEM((half,), jnp.float32),
            pltpu.SemaphoreType.DMA,
        ],
    )
    def gather_kernel(xt_hbm, tt_hbm, tail_hbm, out_hbm, rm_hbm, flat_hbm,
                      buf, tbuf, idx_v, val_v, sem):
        c = lax.axis_index("c")
        s = lax.axis_index("s")
        d_base = pl.multiple_of(c * d_per_c, d_per_c)

        def detile(k, _):
            g = s + k * ns
            off = g * chunk_i
            pltpu.sync_copy(tt_hbm.at[pl.ds(d_base, d_per_c),
                                      pl.ds(off, chunk_i)],
                            rm_hbm.at[pl.ds(d_base, d_per_c),
                                      pl.ds(off, chunk_i)])
            return ()

        n_k = jnp.where(s < n_full % ns, n_full // ns + 1, n_full // ns)
        lax.fori_loop(0, n_k, detile, (), unroll=False)

        @pl.when(s == ns - 1)
        def _():
            pltpu.sync_copy(tt_hbm.at[pl.ds(d_base, d_per_c),
                                      pl.ds(extra_i, extra_sz)],
                            rm_hbm.at[pl.ds(d_base, d_per_c),
                                      pl.ds(extra_i, extra_sz)])

        plsc.subcore_barrier()

        n_row_chunks = (n_full * chunk_i + extra_sz) // chunk_i + 1
        row_units = d_per_c * n_row_chunks

        def flatten(k, _):
            u = s + k * ns
            dr = u // n_row_chunks
            j = u % n_row_chunks
            d = c * d_per_c + dr
            off = j * chunk_i
            @pl.when(j < n_row_chunks - 1)
            def _():
                pltpu.sync_copy(rm_hbm.at[d, pl.ds(off, chunk_i)], buf)
                pltpu.sync_copy(buf, flat_hbm.at[pl.ds(d * vocab + off,
                                                       chunk_i)])

            @pl.when(j == n_row_chunks - 1)
            def _():
                pltpu.sync_copy(rm_hbm.at[d, pl.ds(extra_i, extra_sz)],
                                buf.at[pl.ds(0, extra_sz)])
                pltpu.sync_copy(buf.at[pl.ds(0, extra_sz)],
                                flat_hbm.at[pl.ds(d * vocab + extra_i,
                                                  extra_sz)])
                pltpu.sync_copy(tail_hbm.at[pl.ds(d * tail_sz, tail_sz)], tbuf)
                pltpu.sync_copy(tbuf, flat_hbm.at[pl.ds(d * vocab + tail_i,
                                                        tail_sz)])
            return ()

        n_u = jnp.where(s < row_units % ns, row_units // ns + 1,
                        row_units // ns)
        lax.fori_loop(0, n_u, flatten, (), unroll=False)
        plsc.subcore_barrier()

        def unit(k, _):
            ug = s * u_per_w + k
            f = ug // (d_per_c * 2)
            r = ug % (d_per_c * 2)
            d = c * d_per_c + r // 2
            h = r % 2
            pltpu.sync_copy(xt_hbm.at[f, pl.ds(h * half, half)], idx_v)
            seg = flat_hbm.at[pl.ds(d * vocab, vocab)]
            pltpu.async_copy(seg.at[idx_v], val_v, sem).wait()
            pltpu.sync_copy(val_v, out_hbm.at[f, d, pl.ds(h * half, half)])
            return ()

        lax.fori_loop(0, u_per_w, unit, (), unroll=False)

    return gather_kernel


def kernel(x, table):
    batch = x.shape[0]
    vocab = table.shape[0]
    xt = x.T[:FIELDS_USED].astype(jnp.int32)
    tt = table.T
    tail_i = (vocab // LANE) * LANE
    tail_flat = table[tail_i:].T.reshape(-1)
    out = _make_gather(batch, vocab)(xt, tt, tail_flat)
    return out.transpose(2, 0, 1)


# speedup vs baseline: 1.0297x; 1.0297x over previous
import functools

import jax
import jax.numpy as jnp
from jax import lax
from jax.experimental import pallas as pl
from jax.experimental.pallas import tpu as pltpu
from jax.experimental.pallas import tpu_sc as plsc

EMBED = 16
FIELDS_USED = 9
LANE = 128


@functools.cache
def _make_gather(batch: int, vocab: int):
    nc, ns = 2, 16
    d_per_c = EMBED // nc
    units = FIELDS_USED * d_per_c * 2
    u_per_w = units // ns
    half = batch // 2

    blk_i = 64 * LANE
    n_blk = vocab // blk_i
    extra_i = n_blk * blk_i
    extra_sz = (vocab - extra_i) // LANE * LANE
    tail_i = extra_i + extra_sz
    tail_sz = vocab - tail_i

    ch = 32768
    n_ch = tail_i // ch
    rem_i = n_ch * ch
    rem_sz = tail_i - rem_i
    row_units = d_per_c * n_ch
    per_a = -(-n_blk // ns)
    per_b = -(-row_units // ns)
    mesh = plsc.VectorSubcoreMesh(core_axis_name="c", subcore_axis_name="s")

    @functools.partial(
        pl.kernel,
        mesh=mesh,
        out_type=jax.ShapeDtypeStruct((FIELDS_USED, EMBED, batch), jnp.float32),
        scratch_types=[
            pltpu.HBM((EMBED, vocab), jnp.float32),
            pltpu.HBM((EMBED * vocab,), jnp.float32),
            pltpu.VMEM((ch,), jnp.float32),
            pltpu.VMEM((ch,), jnp.float32),
            pltpu.VMEM((tail_sz,), jnp.float32),
            pltpu.VMEM((half,), jnp.int32),
            pltpu.VMEM((half,), jnp.int32),
            pltpu.VMEM((half,), jnp.float32),
            pltpu.VMEM((half,), jnp.float32),
            pltpu.SemaphoreType.DMA,
            pltpu.SemaphoreType.DMA,
            pltpu.SemaphoreType.DMA,
            pltpu.SemaphoreType.DMA,
            pltpu.SemaphoreType.DMA,
            pltpu.SemaphoreType.DMA,
        ],
    )
    def gather_kernel(xt_hbm, tt_hbm, tail_hbm, out_hbm, rm_hbm, flat_hbm,
                      b0, b1, tbuf, i0, i1, v0, v1,
                      sem_a, sem_l, sem_s, sem_i, sem_g, sem_o):
        c = lax.axis_index("c")
        s = lax.axis_index("s")
        d_base = pl.multiple_of(c * d_per_c, d_per_c)
        bufs = (b0, b1)
        ibufs = (i0, i1)
        vbufs = (v0, v1)

        def a_copy(k):
            g = s + k * ns
            src = tt_hbm.at[pl.ds(d_base, d_per_c), pl.ds(g * blk_i, blk_i)]
            dst = rm_hbm.at[pl.ds(d_base, d_per_c), pl.ds(g * blk_i, blk_i)]
            return src, dst

        for k in range(per_a):
            @pl.when(s + k * ns < n_blk)
            def _(k=k):
                src, dst = a_copy(k)
                pltpu.async_copy(src, dst, sem_a)

        @pl.when(s == ns - 1)
        def _():
            pltpu.sync_copy(
                tt_hbm.at[pl.ds(d_base, d_per_c), pl.ds(extra_i, extra_sz)],
                rm_hbm.at[pl.ds(d_base, d_per_c), pl.ds(extra_i, extra_sz)])

        for k in range(per_a):
            @pl.when(s + k * ns < n_blk)
            def _(k=k):
                src, dst = a_copy(k)
                pltpu.make_async_copy(src, dst, sem_a).wait()

        plsc.subcore_barrier()

        def b_unit(k):
            u = s + k * ns
            dr = u // n_ch
            j = u % n_ch
            d = c * d_per_c + dr
            src = rm_hbm.at[d, pl.ds(j * ch, ch)]
            dst = flat_hbm.at[pl.ds(d * vocab + j * ch, ch)]
            return src, dst

        store_handles = [None] * per_b
        for k in range(per_b):
            src, dst = b_unit(k)
            buf = bufs[k % 2]
            if k >= 2:
                store_handles[k - 2].wait()
            pltpu.async_copy(src, buf, sem_l).wait()
            store_handles[k] = pltpu.async_copy(buf, dst, sem_s)
        for k in range(max(per_b - 2, 0), per_b):
            store_handles[k].wait()

        @pl.when(s < d_per_c)
        def _():
            d = c * d_per_c + s
            pltpu.sync_copy(rm_hbm.at[d, pl.ds(rem_i, rem_sz)],
                            b0.at[pl.ds(0, rem_sz)])
            pltpu.sync_copy(b0.at[pl.ds(0, rem_sz)],
                            flat_hbm.at[pl.ds(d * vocab + rem_i, rem_sz)])
            pltpu.sync_copy(tail_hbm.at[pl.ds(d * tail_sz, tail_sz)], tbuf)
            pltpu.sync_copy(tbuf,
                            flat_hbm.at[pl.ds(d * vocab + tail_i, tail_sz)])

        plsc.subcore_barrier()

        def c_unit(k):
            ug = s * u_per_w + k
            f = ug // (d_per_c * 2)
            r = ug % (d_per_c * 2)
            d = c * d_per_c + r // 2
            h = r % 2
            return f, d, h

        def idx_src(k):
            f, _, h = c_unit(k)
            return xt_hbm.at[f, pl.ds(h * half, half)]

        idx_handle = pltpu.async_copy(idx_src(0), ibufs[0], sem_i)
        out_handles = [None] * u_per_w
        for k in range(u_per_w):
            f, d, h = c_unit(k)
            iv = ibufs[k % 2]
            vv = vbufs[k % 2]
            idx_handle.wait()
            seg = flat_hbm.at[pl.ds(d * vocab, vocab)]
            gather = pltpu.async_copy(seg.at[iv], vv, sem_g)
            if k + 1 < u_per_w:
                idx_handle = pltpu.async_copy(idx_src(k + 1),
                                              ibufs[(k + 1) % 2], sem_i)
            gather.wait()
            dst = out_hbm.at[f, d, pl.ds(h * half, half)]
            out_handles[k] = pltpu.async_copy(vv, dst, sem_o)
            if k >= 1:
                out_handles[k - 1].wait()
        out_handles[u_per_w - 1].wait()

    return gather_kernel


def kernel(x, table):
    batch = x.shape[0]
    vocab = table.shape[0]
    xt = x.T[:FIELDS_USED].astype(jnp.int32)
    tt = table.T
    tail_i = (vocab // LANE) * LANE
    tail_flat = table[tail_i:].T.reshape(-1)
    out = _make_gather(batch, vocab)(xt, tt, tail_flat)
    return out.transpose(2, 0, 1)


# speedup vs baseline: 3.7540x; 3.6458x over previous
import functools

import jax
import jax.numpy as jnp
from jax import lax
from jax.experimental import pallas as pl
from jax.experimental.pallas import tpu as pltpu
from jax.experimental.pallas import tpu_sc as plsc

EMBED = 16
FIELDS_USED = 9


@functools.cache
def _make_gather(batch: int):
    nc, ns = 2, 16
    nw = nc * ns
    b_total = batch * FIELDS_USED
    assert b_total % nw == 0
    b_per_w = b_total // nw
    mesh = plsc.VectorSubcoreMesh(core_axis_name="c", subcore_axis_name="s")

    @functools.partial(
        pl.kernel,
        mesh=mesh,
        out_type=jax.ShapeDtypeStruct((b_total, EMBED), jnp.float32),
        scratch_types=[
            pltpu.VMEM((b_per_w,), jnp.int32),
            pltpu.VMEM((b_per_w, EMBED), jnp.float32),
            pltpu.SemaphoreType.DMA,
        ],
        compiler_params=pltpu.CompilerParams(use_tc_tiling_on_sc=False),
    )
    def gather_kernel(idx_hbm, table_hbm, out_hbm, idx_v, rows_v, sem):
        wid = lax.axis_index("s") * nc + lax.axis_index("c")
        base = wid * b_per_w
        pltpu.sync_copy(idx_hbm.at[pl.ds(base, b_per_w)], idx_v)
        pltpu.async_copy(table_hbm.at[idx_v], rows_v, sem).wait()
        pltpu.sync_copy(rows_v, out_hbm.at[pl.ds(base, b_per_w)])

    return gather_kernel


def kernel(x, table):
    batch = x.shape[0]
    idx = x[:, :FIELDS_USED].reshape(-1).astype(jnp.int32)
    out = _make_gather(batch)(idx, table)
    return out.reshape(batch, FIELDS_USED, EMBED)
